# baseline (speedup 1.0000x reference)
import functools

import jax
import jax.numpy as jnp
from jax import lax
from jax.experimental import pallas as pl
from jax.experimental.pallas import tpu as pltpu
from jax.experimental.pallas import tpu_sc as plsc

NSAMPLE = 16
KH = NSAMPLE // 2
LEAKY = 0.1
TN1 = 256
RT = 256
DG = 128
GCH = 128


def _leaky(x):
    return jnp.where(x >= 0, x, LEAKY * x)


def _knn_body(knn1_ref, knn2_ref, xyz1_ref, xyz2_ref, idx_ref):
    b = pl.program_id(0)
    n2 = knn2_ref.shape[2]
    k1 = knn1_ref[0]
    k2 = knn2_ref[0]
    dot = lax.dot_general(k1, k2, (((0,), (0,)), ((), ())),
                          preferred_element_type=jnp.float32)
    csq = jnp.sum(k2 * k2, axis=0, keepdims=True)
    s = -dot * lax.rsqrt(csq + 1e-8)
    col = lax.broadcasted_iota(jnp.int32, (TN1, n2), 1)

    colf = col.astype(jnp.float32)

    def rowmin(x):
        t = jnp.min(x.reshape(TN1, n2 // 128, 128), axis=2)
        return jnp.min(t, axis=1, keepdims=True)

    def top8(s):
        picks = []
        m = rowmin(s)
        for _ in range(KH):
            am = rowmin(jnp.where(s == m, colf, jnp.float32(3e38)))
            picks.append(am.astype(jnp.int32).reshape(TN1, 1))
            m = rowmin(jnp.where(s <= m, jnp.float32(3e38), s))
        return picks

    picks = top8(s)
    x1 = xyz1_ref[0]
    x2 = xyz2_ref[0]
    d = lax.dot_general(x1, x2, (((0,), (0,)), ((), ())),
                        preferred_element_type=jnp.float32)
    s2 = jnp.sum(x2 * x2, axis=0, keepdims=True) - 2.0 * d
    picks += top8(s2)
    idx_ref[0] = jnp.concatenate(picks, axis=1) + b * n2


def _knn_pallas(knn1, knn2, xyz1, xyz2):
    b, c, n1 = knn1.shape
    n2 = knn2.shape[2]
    return pl.pallas_call(
        _knn_body,
        grid=(b, n1 // TN1),
        in_specs=[
            pl.BlockSpec((1, c, TN1), lambda i, t: (i, 0, t)),
            pl.BlockSpec((1, c, n2), lambda i, t: (i, 0, 0)),
            pl.BlockSpec((1, 3, TN1), lambda i, t: (i, 0, t)),
            pl.BlockSpec((1, 3, n2), lambda i, t: (i, 0, 0)),
        ],
        out_specs=pl.BlockSpec((1, TN1, NSAMPLE), lambda i, t: (i, t, 0)),
        out_shape=jax.ShapeDtypeStruct((b, n1, NSAMPLE), jnp.int32),
    )(knn1, knn2, xyz1, xyz2)


def _gather_pallas(table, idx_flat):
    tot = idx_flat.shape[0]
    nwork = 32
    per = tot // nwork
    nch = per // GCH
    mesh = plsc.VectorSubcoreMesh(core_axis_name="c", subcore_axis_name="s")

    @functools.partial(
        pl.kernel,
        out_type=jax.ShapeDtypeStruct((tot, DG), jnp.float32),
        scratch_types=[
            pltpu.VMEM((GCH,), jnp.int32),
            pltpu.VMEM((GCH, DG), jnp.float32),
            pltpu.SemaphoreType.DMA,
        ],
        mesh=mesh,
    )
    def gk(table_hbm, idx_hbm, out_hbm, idx_v, rows_v, sem):
        wid = lax.axis_index("s") * 2 + lax.axis_index("c")
        for ch in range(nch):
            base = wid * per + ch * GCH
            pltpu.sync_copy(idx_hbm.at[pl.ds(base, GCH)], idx_v)
            pltpu.async_copy(table_hbm.at[idx_v], rows_v, sem).wait()
            pltpu.sync_copy(rows_v, out_hbm.at[pl.ds(base, GCH)])

    return gk(table, idx_flat)


_PARAM_ORDER = (
    'W_r0', 'b_r0', 'W_r1', 'b_r1', 'W_r2', 'b_r2',
    'W_z0', 'b_z0', 'W_z1', 'b_z1', 'W_z2', 'b_z2',
    'W_h0', 'b_h0', 'W_h1', 'b_h1', 'W_h2', 'b_h2',
    'fuse_r', 'fuse_r_o', 'fuse_z', 'fuse_r_2', 'fuse_r_o_2', 'fuse_z_2',
)


def _mm(x, w):
    return lax.dot_general(x, w, (((1,), (1,)), ((), ())),
                           preferred_element_type=jnp.float32)


def _gru_body(g_ref, p1t_ref, x1t_ref,
              wr0, br0, wr1, br1, wr2, br2,
              wz0, bz0, wz1, bz1, wz2, bz2,
              wh0, bh0, wh1, bh1, wh2, bh2,
              fr, fro, fz, fr2, fro2, fz2,
              out_ref):
    m = RT * NSAMPLE
    g = g_ref[0]
    nx = g[:, 0:3]
    feat = g[:, 3:3 + 64]
    x1 = x1t_ref[0]
    p1 = p1t_ref[0]
    x1b = jnp.broadcast_to(x1[:, None, :], (RT, NSAMPLE, 3)).reshape(m, 3)
    d3 = nx - x1b

    gcat = _mm(feat, jnp.concatenate([fr2[0], fro2[0], fz2[0]], axis=0))
    l0 = _mm(d3, jnp.concatenate([wr0[0], wz0[0], wh0[0]], axis=0))
    p1cat = _mm(p1, jnp.concatenate([fr[0], fz[0], fro[0]], axis=0))

    rz = l0[:, 0:128] + jnp.concatenate([br0[0], bz0[0]], axis=1) \
        + gcat[:, 0:128]
    rz = rz.reshape(RT, NSAMPLE, 128) + p1cat[:, None, 0:128]
    rz = _leaky(rz).reshape(m, 128)
    wrz1 = jnp.concatenate([
        jnp.concatenate([wr1[0], jnp.zeros((64, 64), jnp.float32)], axis=1),
        jnp.concatenate([jnp.zeros((64, 64), jnp.float32), wz1[0]], axis=1),
    ], axis=0)
    rz = _mm(rz, wrz1) + jnp.concatenate([br1[0], bz1[0]], axis=1)
    r = _leaky(rz[:, 0:64])
    z = _leaky(rz[:, 64:128])
    r = jax.nn.sigmoid(_mm(r, wr2[0]) + br2[0])
    z = jnp.max(z.reshape(RT, NSAMPLE, 64), axis=1)
    z = jax.nn.sigmoid(_mm(z, wz2[0]) + bz2[0])

    p1_exp = (r.reshape(RT, NSAMPLE, 64) * p1cat[:, None, 128:192]).reshape(m, 64)
    h = l0[:, 128:192] + bh0[0] + p1_exp + gcat[:, 128:192]
    h = _leaky(h)
    h = _leaky(_mm(h, wh1[0]) + bh1[0])
    h = jnp.max(h.reshape(RT, NSAMPLE, 64), axis=1)
    h = _leaky(_mm(h, wh2[0]) + bh2[0])

    out_ref[0] = (1.0 - z) * p1 + z * h


def _gru_pallas(g3, p1t, x1t, params):
    b, n1, c = p1t.shape
    wargs = []
    wspecs = []
    for name in _PARAM_ORDER:
        w = params[name]
        w2 = w.reshape(1, *w.shape) if w.ndim == 2 else w.reshape(1, 1, w.shape[0])
        wargs.append(w2)
        wspecs.append(pl.BlockSpec(w2.shape, lambda i, t: (0, 0, 0)))
    return pl.pallas_call(
        _gru_body,
        grid=(b, n1 // RT),
        in_specs=[
            pl.BlockSpec((1, RT * NSAMPLE, DG), lambda i, t: (i, t, 0)),
            pl.BlockSpec((1, RT, c), lambda i, t: (i, t, 0)),
            pl.BlockSpec((1, RT, 3), lambda i, t: (i, t, 0)),
        ] + wspecs,
        out_specs=pl.BlockSpec((1, RT, c), lambda i, t: (i, t, 0)),
        out_shape=jax.ShapeDtypeStruct((b, n1, c), jnp.float32),
    )(g3, p1t, x1t, *wargs)


def kernel(xyz1, xyz2, points1, points2, knn1, knn2, params):
    b, _, n1 = xyz1.shape
    n2 = xyz2.shape[2]
    c = points1.shape[1]
    idx = _knn_pallas(knn1, knn2, xyz1, xyz2)
    xyz2t = jnp.transpose(xyz2, (0, 2, 1)).reshape(b * n2, 3)
    p2t = jnp.transpose(points2, (0, 2, 1)).reshape(b * n2, c)
    pad = jnp.zeros((b * n2, DG - 3 - c), jnp.float32)
    table = jnp.concatenate([xyz2t, p2t, pad], axis=1)
    gathered = _gather_pallas(table, idx.reshape(b * n1 * NSAMPLE))
    g3 = gathered.reshape(b, n1 * NSAMPLE, DG)
    p1t = jnp.transpose(points1, (0, 2, 1))
    x1t = jnp.transpose(xyz1, (0, 2, 1))
    out = _gru_pallas(g3, p1t, x1t, params)
    return jnp.transpose(out, (0, 2, 1))


# speedup vs baseline: 2.9838x; 2.9838x over previous
import functools

import jax
import jax.numpy as jnp
from jax import lax
from jax.experimental import pallas as pl
from jax.experimental.pallas import tpu as pltpu
from jax.experimental.pallas import tpu_sc as plsc

NSAMPLE = 16
KH = NSAMPLE // 2
LEAKY = 0.1
TN1 = 256
RT = 256
DG = 128
GCH = 128


def _leaky(x):
    return jnp.where(x >= 0, x, LEAKY * x)


def _knn_body(knn1_ref, knn2_ref, xyz1_ref, xyz2_ref, idx_ref):
    b = pl.program_id(0)
    n2 = knn2_ref.shape[2]
    k1 = knn1_ref[0]
    k2 = knn2_ref[0]
    dot = lax.dot_general(k1, k2, (((0,), (0,)), ((), ())),
                          preferred_element_type=jnp.float32)
    csq = jnp.sum(k2 * k2, axis=0, keepdims=True)
    s = -dot * lax.rsqrt(csq + 1e-8)
    col = lax.broadcasted_iota(jnp.int32, (TN1, n2), 1)

    colf = col.astype(jnp.float32)

    def rowmin(x):
        return jnp.min(x, axis=1, keepdims=True)

    def top8(s):
        picks = []
        m = rowmin(s)
        for _ in range(KH):
            am = rowmin(jnp.where(s == m, colf, jnp.float32(3e38)))
            picks.append(am.astype(jnp.int32).reshape(TN1, 1))
            m = rowmin(jnp.where(s <= m, jnp.float32(3e38), s))
        return picks

    picks = top8(s)
    x1 = xyz1_ref[0]
    x2 = xyz2_ref[0]
    d = lax.dot_general(x1, x2, (((0,), (0,)), ((), ())),
                        preferred_element_type=jnp.float32)
    s2 = jnp.sum(x2 * x2, axis=0, keepdims=True) - 2.0 * d
    picks += top8(s2)
    idx_ref[0] = jnp.concatenate(picks, axis=1) + b * n2


def _knn_pallas(knn1, knn2, xyz1, xyz2):
    b, c, n1 = knn1.shape
    n2 = knn2.shape[2]
    return pl.pallas_call(
        _knn_body,
        grid=(b, n1 // TN1),
        in_specs=[
            pl.BlockSpec((1, c, TN1), lambda i, t: (i, 0, t)),
            pl.BlockSpec((1, c, n2), lambda i, t: (i, 0, 0)),
            pl.BlockSpec((1, 3, TN1), lambda i, t: (i, 0, t)),
            pl.BlockSpec((1, 3, n2), lambda i, t: (i, 0, 0)),
        ],
        out_specs=pl.BlockSpec((1, TN1, NSAMPLE), lambda i, t: (i, t, 0)),
        out_shape=jax.ShapeDtypeStruct((b, n1, NSAMPLE), jnp.int32),
    )(knn1, knn2, xyz1, xyz2)


def _gather_pallas(table, idx_flat):
    tot = idx_flat.shape[0]
    nwork = 32
    per = tot // nwork
    nch = per // GCH
    mesh = plsc.VectorSubcoreMesh(core_axis_name="c", subcore_axis_name="s")

    @functools.partial(
        pl.kernel,
        out_type=jax.ShapeDtypeStruct((tot, DG), jnp.float32),
        scratch_types=[
            pltpu.VMEM((GCH,), jnp.int32),
            pltpu.VMEM((GCH, DG), jnp.float32),
            pltpu.SemaphoreType.DMA,
        ],
        mesh=mesh,
    )
    def gk(table_hbm, idx_hbm, out_hbm, idx_v, rows_v, sem):
        wid = lax.axis_index("s") * 2 + lax.axis_index("c")
        for ch in range(nch):
            base = wid * per + ch * GCH
            pltpu.sync_copy(idx_hbm.at[pl.ds(base, GCH)], idx_v)
            pltpu.async_copy(table_hbm.at[idx_v], rows_v, sem).wait()
            pltpu.sync_copy(rows_v, out_hbm.at[pl.ds(base, GCH)])

    return gk(table, idx_flat)


_PARAM_ORDER = (
    'W_r0', 'b_r0', 'W_r1', 'b_r1', 'W_r2', 'b_r2',
    'W_z0', 'b_z0', 'W_z1', 'b_z1', 'W_z2', 'b_z2',
    'W_h0', 'b_h0', 'W_h1', 'b_h1', 'W_h2', 'b_h2',
    'fuse_r', 'fuse_r_o', 'fuse_z', 'fuse_r_2', 'fuse_r_o_2', 'fuse_z_2',
)


def _mm(x, w):
    return lax.dot_general(x, w, (((1,), (1,)), ((), ())),
                           preferred_element_type=jnp.float32)


def _gru_body(g_ref, p1t_ref, x1t_ref,
              wr0, br0, wr1, br1, wr2, br2,
              wz0, bz0, wz1, bz1, wz2, bz2,
              wh0, bh0, wh1, bh1, wh2, bh2,
              fr, fro, fz, fr2, fro2, fz2,
              out_ref):
    m = RT * NSAMPLE
    g = g_ref[0]
    nx = g[:, 0:3]
    feat = g[:, 3:3 + 64]
    x1 = x1t_ref[0]
    p1 = p1t_ref[0]
    x1b = jnp.broadcast_to(x1[:, None, :], (RT, NSAMPLE, 3)).reshape(m, 3)
    d3 = nx - x1b

    gcat = _mm(feat, jnp.concatenate([fr2[0], fro2[0], fz2[0]], axis=0))
    l0 = _mm(d3, jnp.concatenate([wr0[0], wz0[0], wh0[0]], axis=0))
    p1cat = _mm(p1, jnp.concatenate([fr[0], fz[0], fro[0]], axis=0))

    rz = l0[:, 0:128] + jnp.concatenate([br0[0], bz0[0]], axis=1) \
        + gcat[:, 0:128]
    rz = rz.reshape(RT, NSAMPLE, 128) + p1cat[:, None, 0:128]
    rz = _leaky(rz).reshape(m, 128)
    wrz1 = jnp.concatenate([
        jnp.concatenate([wr1[0], jnp.zeros((64, 64), jnp.float32)], axis=1),
        jnp.concatenate([jnp.zeros((64, 64), jnp.float32), wz1[0]], axis=1),
    ], axis=0)
    rz = _mm(rz, wrz1) + jnp.concatenate([br1[0], bz1[0]], axis=1)
    r = _leaky(rz[:, 0:64])
    z = _leaky(rz[:, 64:128])
    r = jax.nn.sigmoid(_mm(r, wr2[0]) + br2[0])
    z = jnp.max(z.reshape(RT, NSAMPLE, 64), axis=1)
    z = jax.nn.sigmoid(_mm(z, wz2[0]) + bz2[0])

    p1_exp = (r.reshape(RT, NSAMPLE, 64) * p1cat[:, None, 128:192]).reshape(m, 64)
    h = l0[:, 128:192] + bh0[0] + p1_exp + gcat[:, 128:192]
    h = _leaky(h)
    h = _leaky(_mm(h, wh1[0]) + bh1[0])
    h = jnp.max(h.reshape(RT, NSAMPLE, 64), axis=1)
    h = _leaky(_mm(h, wh2[0]) + bh2[0])

    out_ref[0] = (1.0 - z) * p1 + z * h


def _gru_pallas(g3, p1t, x1t, params):
    b, n1, c = p1t.shape
    wargs = []
    wspecs = []
    for name in _PARAM_ORDER:
        w = params[name]
        w2 = w.reshape(1, *w.shape) if w.ndim == 2 else w.reshape(1, 1, w.shape[0])
        wargs.append(w2)
        wspecs.append(pl.BlockSpec(w2.shape, lambda i, t: (0, 0, 0)))
    return pl.pallas_call(
        _gru_body,
        grid=(b, n1 // RT),
        in_specs=[
            pl.BlockSpec((1, RT * NSAMPLE, DG), lambda i, t: (i, t, 0)),
            pl.BlockSpec((1, RT, c), lambda i, t: (i, t, 0)),
            pl.BlockSpec((1, RT, 3), lambda i, t: (i, t, 0)),
        ] + wspecs,
        out_specs=pl.BlockSpec((1, RT, c), lambda i, t: (i, t, 0)),
        out_shape=jax.ShapeDtypeStruct((b, n1, c), jnp.float32),
    )(g3, p1t, x1t, *wargs)


def kernel(xyz1, xyz2, points1, points2, knn1, knn2, params):
    b, _, n1 = xyz1.shape
    n2 = xyz2.shape[2]
    c = points1.shape[1]
    xyz2t = jnp.transpose(xyz2, (0, 2, 1)).reshape(b * n2, 3)
    p2t = jnp.transpose(points2, (0, 2, 1)).reshape(b * n2, c)
    pad = jnp.zeros((b * n2, DG - 3 - c), jnp.float32)
    table = jnp.concatenate([xyz2t, p2t, pad], axis=1)
    p1t = jnp.transpose(points1, (0, 2, 1))
    x1t = jnp.transpose(xyz1, (0, 2, 1))
    idxs = [_knn_pallas(knn1[i:i + 1], knn2[i:i + 1],
                        xyz1[i:i + 1], xyz2[i:i + 1]) + i * n2
            for i in range(b)]
    gath = [_gather_pallas(table, idxs[i].reshape(n1 * NSAMPLE))
            for i in range(b)]
    outs = [_gru_pallas(gath[i].reshape(1, n1 * NSAMPLE, DG),
                        p1t[i:i + 1], x1t[i:i + 1], params)
            for i in range(b)]
    return jnp.transpose(jnp.concatenate(outs, axis=0), (0, 2, 1))


# speedup vs baseline: 3.0027x; 1.0063x over previous
import functools

import jax
import jax.numpy as jnp
from jax import lax
from jax.experimental import pallas as pl
from jax.experimental.pallas import tpu as pltpu
from jax.experimental.pallas import tpu_sc as plsc

NSAMPLE = 16
KH = NSAMPLE // 2
LEAKY = 0.1
TN1 = 256
RT = 256
DG = 128
GCH = 128


def _leaky(x):
    return jnp.where(x >= 0, x, LEAKY * x)


def _knn_body(knn1_ref, knn2_ref, xyz1_ref, xyz2_ref, idx_ref):
    b = pl.program_id(0)
    n2 = knn2_ref.shape[2]
    k1 = knn1_ref[0]
    k2 = knn2_ref[0]
    dot = lax.dot_general(k1, k2, (((0,), (0,)), ((), ())),
                          preferred_element_type=jnp.float32)
    csq = jnp.sum(k2 * k2, axis=0, keepdims=True)
    s = -dot * lax.rsqrt(csq + 1e-8)
    col = lax.broadcasted_iota(jnp.int32, (TN1, n2), 1)

    colf = col.astype(jnp.float32)

    def rowmin(x):
        return jnp.min(x, axis=1, keepdims=True)

    def top8(s):
        picks = []
        m = rowmin(s)
        prev = jnp.zeros((TN1, 1), jnp.float32)
        for _ in range(KH):
            le = s <= m
            cur = jnp.sum(jnp.where(le, colf, 0.0), axis=1, keepdims=True)
            picks.append(jnp.clip((cur - prev).astype(jnp.int32), 0, n2 - 1))
            prev = cur
            m = rowmin(jnp.where(le, jnp.float32(3e38), s))
        return picks

    picks = top8(s)
    x1 = xyz1_ref[0]
    x2 = xyz2_ref[0]
    d = lax.dot_general(x1, x2, (((0,), (0,)), ((), ())),
                        preferred_element_type=jnp.float32)
    s2 = jnp.sum(x2 * x2, axis=0, keepdims=True) - 2.0 * d
    picks += top8(s2)
    idx_ref[0] = jnp.concatenate(picks, axis=1) + b * n2


def _knn_pallas(knn1, knn2, xyz1, xyz2):
    b, c, n1 = knn1.shape
    n2 = knn2.shape[2]
    return pl.pallas_call(
        _knn_body,
        grid=(b, n1 // TN1),
        in_specs=[
            pl.BlockSpec((1, c, TN1), lambda i, t: (i, 0, t)),
            pl.BlockSpec((1, c, n2), lambda i, t: (i, 0, 0)),
            pl.BlockSpec((1, 3, TN1), lambda i, t: (i, 0, t)),
            pl.BlockSpec((1, 3, n2), lambda i, t: (i, 0, 0)),
        ],
        out_specs=pl.BlockSpec((1, TN1, NSAMPLE), lambda i, t: (i, t, 0)),
        out_shape=jax.ShapeDtypeStruct((b, n1, NSAMPLE), jnp.int32),
    )(knn1, knn2, xyz1, xyz2)


def _gather_pallas(table, idx_flat):
    tot = idx_flat.shape[0]
    nwork = 32
    per = tot // nwork
    nch = per // GCH
    mesh = plsc.VectorSubcoreMesh(core_axis_name="c", subcore_axis_name="s")

    @functools.partial(
        pl.kernel,
        out_type=jax.ShapeDtypeStruct((tot, DG), jnp.float32),
        scratch_types=[
            pltpu.VMEM((GCH,), jnp.int32),
            pltpu.VMEM((GCH, DG), jnp.float32),
            pltpu.SemaphoreType.DMA,
        ],
        mesh=mesh,
    )
    def gk(table_hbm, idx_hbm, out_hbm, idx_v, rows_v, sem):
        wid = lax.axis_index("s") * 2 + lax.axis_index("c")
        for ch in range(nch):
            base = wid * per + ch * GCH
            pltpu.sync_copy(idx_hbm.at[pl.ds(base, GCH)], idx_v)
            pltpu.async_copy(table_hbm.at[idx_v], rows_v, sem).wait()
            pltpu.sync_copy(rows_v, out_hbm.at[pl.ds(base, GCH)])

    return gk(table, idx_flat)


_PARAM_ORDER = (
    'W_r0', 'b_r0', 'W_r1', 'b_r1', 'W_r2', 'b_r2',
    'W_z0', 'b_z0', 'W_z1', 'b_z1', 'W_z2', 'b_z2',
    'W_h0', 'b_h0', 'W_h1', 'b_h1', 'W_h2', 'b_h2',
    'fuse_r', 'fuse_r_o', 'fuse_z', 'fuse_r_2', 'fuse_r_o_2', 'fuse_z_2',
)


def _mm(x, w):
    return lax.dot_general(x, w, (((1,), (1,)), ((), ())),
                           preferred_element_type=jnp.float32)


def _gru_body(g_ref, p1t_ref, x1t_ref,
              wr0, br0, wr1, br1, wr2, br2,
              wz0, bz0, wz1, bz1, wz2, bz2,
              wh0, bh0, wh1, bh1, wh2, bh2,
              fr, fro, fz, fr2, fro2, fz2,
              out_ref):
    m = RT * NSAMPLE
    g = g_ref[0]
    nx = g[:, 0:3]
    feat = g[:, 3:3 + 64]
    x1 = x1t_ref[0]
    p1 = p1t_ref[0]
    x1b = jnp.broadcast_to(x1[:, None, :], (RT, NSAMPLE, 3)).reshape(m, 3)
    d3 = nx - x1b

    gcat = _mm(feat, jnp.concatenate([fr2[0], fro2[0], fz2[0]], axis=0))
    l0 = _mm(d3, jnp.concatenate([wr0[0], wz0[0], wh0[0]], axis=0))
    p1cat = _mm(p1, jnp.concatenate([fr[0], fz[0], fro[0]], axis=0))

    rz = l0[:, 0:128] + jnp.concatenate([br0[0], bz0[0]], axis=1) \
        + gcat[:, 0:128]
    rz = rz.reshape(RT, NSAMPLE, 128) + p1cat[:, None, 0:128]
    rz = _leaky(rz).reshape(m, 128)
    wrz1 = jnp.concatenate([
        jnp.concatenate([wr1[0], jnp.zeros((64, 64), jnp.float32)], axis=1),
        jnp.concatenate([jnp.zeros((64, 64), jnp.float32), wz1[0]], axis=1),
    ], axis=0)
    rz = _mm(rz, wrz1) + jnp.concatenate([br1[0], bz1[0]], axis=1)
    r = _leaky(rz[:, 0:64])
    z = _leaky(rz[:, 64:128])
    r = jax.nn.sigmoid(_mm(r, wr2[0]) + br2[0])
    z = jnp.max(z.reshape(RT, NSAMPLE, 64), axis=1)
    z = jax.nn.sigmoid(_mm(z, wz2[0]) + bz2[0])

    p1_exp = (r.reshape(RT, NSAMPLE, 64) * p1cat[:, None, 128:192]).reshape(m, 64)
    h = l0[:, 128:192] + bh0[0] + p1_exp + gcat[:, 128:192]
    h = _leaky(h)
    h = _leaky(_mm(h, wh1[0]) + bh1[0])
    h = jnp.max(h.reshape(RT, NSAMPLE, 64), axis=1)
    h = _leaky(_mm(h, wh2[0]) + bh2[0])

    out_ref[0] = (1.0 - z) * p1 + z * h


def _gru_pallas(g3, p1t, x1t, params):
    b, n1, c = p1t.shape
    wargs = []
    wspecs = []
    for name in _PARAM_ORDER:
        w = params[name]
        w2 = w.reshape(1, *w.shape) if w.ndim == 2 else w.reshape(1, 1, w.shape[0])
        wargs.append(w2)
        wspecs.append(pl.BlockSpec(w2.shape, lambda i, t: (0, 0, 0)))
    return pl.pallas_call(
        _gru_body,
        grid=(b, n1 // RT),
        in_specs=[
            pl.BlockSpec((1, RT * NSAMPLE, DG), lambda i, t: (i, t, 0)),
            pl.BlockSpec((1, RT, c), lambda i, t: (i, t, 0)),
            pl.BlockSpec((1, RT, 3), lambda i, t: (i, t, 0)),
        ] + wspecs,
        out_specs=pl.BlockSpec((1, RT, c), lambda i, t: (i, t, 0)),
        out_shape=jax.ShapeDtypeStruct((b, n1, c), jnp.float32),
    )(g3, p1t, x1t, *wargs)


def kernel(xyz1, xyz2, points1, points2, knn1, knn2, params):
    b, _, n1 = xyz1.shape
    n2 = xyz2.shape[2]
    c = points1.shape[1]
    xyz2t = jnp.transpose(xyz2, (0, 2, 1)).reshape(b * n2, 3)
    p2t = jnp.transpose(points2, (0, 2, 1)).reshape(b * n2, c)
    pad = jnp.zeros((b * n2, DG - 3 - c), jnp.float32)
    table = jnp.concatenate([xyz2t, p2t, pad], axis=1)
    p1t = jnp.transpose(points1, (0, 2, 1))
    x1t = jnp.transpose(xyz1, (0, 2, 1))
    idxs = [_knn_pallas(knn1[i:i + 1], knn2[i:i + 1],
                        xyz1[i:i + 1], xyz2[i:i + 1]) + i * n2
            for i in range(b)]
    gath = [_gather_pallas(table, idxs[i].reshape(n1 * NSAMPLE))
            for i in range(b)]
    outs = [_gru_pallas(gath[i].reshape(1, n1 * NSAMPLE, DG),
                        p1t[i:i + 1], x1t[i:i + 1], params)
            for i in range(b)]
    return jnp.transpose(jnp.concatenate(outs, axis=0), (0, 2, 1))
